# baseline (speedup 1.0000x reference)
import functools

import jax
import jax.numpy as jnp
from jax import lax
from jax.experimental import pallas as pl
from jax.experimental.pallas import tpu as pltpu
from jax.experimental.pallas import tpu_sc as plsc

N_PRED = 4096
N_GT = 16384
BM = 1024
CHUNK = 2048

_SC_NC = 2
_SC_NS = 16
_NW = _SC_NC * _SC_NS
_B_PER_W = N_PRED // _NW


def _k1gt_body(a_ref, gt_ref, minsq_ref, idx_ref, cg_ref, gaug_ref):
    i = pl.program_id(0)
    a = a_ref[...]
    a_bf = a.astype(jnp.bfloat16)
    a2 = jnp.sum(a * a, axis=1, keepdims=True)
    cg_ref[...] = lax.broadcasted_iota(jnp.int32, (1, N_GT), 1)
    big = jnp.float32(3.0e38)
    bigi = jnp.int32(2 ** 30)

    @pl.when(i == 0)
    def _():
        g = gt_ref[...]
        b2 = g[0:1] * g[0:1] + g[1:2] * g[1:2] + g[2:3] * g[2:3]
        gaug_ref[0:3, :] = g
        gaug_ref[3:4, :] = -0.5 * b2

    aug1 = jnp.concatenate(
        [a_bf, jnp.ones((BM, 1), jnp.bfloat16)], axis=1)

    gmax = jnp.full((BM, 1), -big, jnp.float32)
    gidx = jnp.zeros((BM, 1), jnp.int32)
    for c in range(N_GT // CHUNK):
        ga = gaug_ref[:, c * CHUNK:(c + 1) * CHUNK]
        m = jnp.dot(aug1, ga.astype(jnp.bfloat16),
                    preferred_element_type=jnp.float32)
        cmax = jnp.max(m, axis=1, keepdims=True)
        csel = jnp.where(m == cmax, cg_ref[:, c * CHUNK:(c + 1) * CHUNK],
                         bigi)
        cidx = jnp.min(csel, axis=1, keepdims=True)
        upd = cmax > gmax
        gmax = jnp.where(upd, cmax, gmax)
        gidx = jnp.where(upd, cidx, gidx)
    idx_ref[...] = gidx
    minsq_ref[...] = jnp.maximum(a2 - 2.0 * gmax, 1e-12)


def _k1self_body(a_ref, pt_ref, minself_ref, cg_ref):
    i = pl.program_id(0)
    a = a_ref[...]
    a_bf = a.astype(jnp.bfloat16)
    a2 = jnp.sum(a * a, axis=1, keepdims=True)
    cg_ref[...] = lax.broadcasted_iota(jnp.int32, (1, N_PRED), 1)
    big = jnp.float32(3.0e38)
    rowg = lax.broadcasted_iota(jnp.int32, (BM, 1), 0) + i * BM

    ms = jnp.full((BM, 1), big, jnp.float32)
    for c in range(N_PRED // CHUNK):
        p = pt_ref[:, c * CHUNK:(c + 1) * CHUNK]
        b2 = p[0:1] * p[0:1] + p[1:2] * p[1:2] + p[2:3] * p[2:3]
        m = jnp.dot(a_bf, p.astype(jnp.bfloat16),
                    preferred_element_type=jnp.float32)
        r = b2 - 2.0 * m
        cg = cg_ref[:, c * CHUNK:(c + 1) * CHUNK]
        r = jnp.where(cg == rowg, big, r)
        ms = jnp.minimum(ms, jnp.min(r, axis=1, keepdims=True))
    minself_ref[...] = jnp.maximum(a2 + ms, 1e-12)


_k1gt = pl.pallas_call(
    _k1gt_body,
    grid=(N_PRED // BM,),
    in_specs=[
        pl.BlockSpec((BM, 3), lambda i: (i, 0)),
        pl.BlockSpec((3, N_GT), lambda i: (0, 0)),
    ],
    out_specs=[
        pl.BlockSpec((BM, 1), lambda i: (i, 0)),
        pl.BlockSpec((BM, 1), lambda i: (i, 0)),
    ],
    out_shape=[
        jax.ShapeDtypeStruct((N_PRED, 1), jnp.float32),
        jax.ShapeDtypeStruct((N_PRED, 1), jnp.int32),
    ],
    scratch_shapes=[pltpu.VMEM((1, N_GT), jnp.int32),
                    pltpu.VMEM((4, N_GT), jnp.float32)],
)

_k1self = pl.pallas_call(
    _k1self_body,
    grid=(N_PRED // BM,),
    in_specs=[
        pl.BlockSpec((BM, 3), lambda i: (i, 0)),
        pl.BlockSpec((3, N_PRED), lambda i: (0, 0)),
    ],
    out_specs=pl.BlockSpec((BM, 1), lambda i: (i, 0)),
    out_shape=jax.ShapeDtypeStruct((N_PRED, 1), jnp.float32),
    scratch_shapes=[pltpu.VMEM((1, N_PRED), jnp.int32)],
)


@functools.lru_cache(maxsize=1)
def _make_sc_gather():
    mesh = plsc.VectorSubcoreMesh(core_axis_name="c", subcore_axis_name="s")

    @functools.partial(
        pl.kernel,
        mesh=mesh,
        out_type=jax.ShapeDtypeStruct((N_PRED, 6), jnp.float32),
        scratch_types=[
            pltpu.VMEM((_B_PER_W,), jnp.int32),
            pltpu.VMEM((_B_PER_W, 6), jnp.float32),
            pltpu.SemaphoreType.DMA,
        ],
        compiler_params=pltpu.CompilerParams(use_tc_tiling_on_sc=False),
    )
    def gather_k(table_hbm, idx_hbm, out_hbm, idx_v, rows_v, sem):
        wid = lax.axis_index("s") * _SC_NC + lax.axis_index("c")
        base = wid * _B_PER_W
        pltpu.sync_copy(idx_hbm.at[pl.ds(base, _B_PER_W)], idx_v)
        pltpu.async_copy(table_hbm.at[idx_v], rows_v, sem).wait()
        pltpu.sync_copy(rows_v, out_hbm.at[pl.ds(base, _B_PER_W)])

    return gather_k


def _k3_body(minsq_ref, minself_ref, pf_ref, g_ref, out_ref):
    minsq = minsq_ref[...]
    minself = minself_ref[...]
    pnt = jnp.transpose(pf_ref[:, 3:6])
    gnt = jnp.transpose(g_ref[:, 3:6])

    attraction = jnp.sum(minsq) / (N_PRED * 3.0)

    d = jnp.sqrt(minself)
    x = 100.0 * (0.3 - d)
    pen = jnp.maximum(x, 0.0) + jnp.log(1.0 + jnp.exp(-jnp.abs(x)))
    repulsion = jnp.sum(pen * pen) / N_PRED

    pn2 = jnp.sum(pnt * pnt, axis=0, keepdims=True)
    gn2 = jnp.sum(gnt * gnt, axis=0, keepdims=True)
    pd = jnp.maximum(jnp.sqrt(pn2), 1e-5)
    gd = jnp.maximum(jnp.sqrt(gn2), 1e-5)
    dot = jnp.sum(pnt * gnt, axis=0, keepdims=True)
    cos = dot / (pd * gd)
    norm_loss = jnp.sum(1.0 - cos) / N_PRED

    out_ref[0, 0] = attraction + repulsion + 10.0 * norm_loss


_k3 = pl.pallas_call(
    _k3_body,
    out_specs=pl.BlockSpec(memory_space=pltpu.SMEM),
    out_shape=jax.ShapeDtypeStruct((1, 1), jnp.float32),
)


def kernel(pred_feat, pred_decoder, input_data, gt_data):
    pp = pred_feat[:, :3]
    gp = gt_data[:, :3]

    minsq, idx = _k1gt(pp, gp.T)
    g = _make_sc_gather()(gt_data, idx.reshape(N_PRED))
    minself = _k1self(pp, pp.T)
    out = _k3(minsq.reshape(32, 128), minself.reshape(32, 128),
              pred_feat, g)
    return out[0, 0]


# speedup vs baseline: 1.0476x; 1.0476x over previous
import functools

import jax
import jax.numpy as jnp
from jax import lax
from jax.experimental import pallas as pl
from jax.experimental.pallas import tpu as pltpu
from jax.experimental.pallas import tpu_sc as plsc

N_PRED = 4096
N_GT = 16384
BM = 512
CHUNK = 2048

_SC_NC = 2
_SC_NS = 16
_NW = _SC_NC * _SC_NS
_B_PER_W = N_PRED // _NW


def _k1_body(a_ref, gt_ref, pt_ref, minsq_ref, idx_ref, minself_ref,
             cg_ref, gaug_ref):
    i = pl.program_id(0)
    a = a_ref[...]
    a_bf = a.astype(jnp.bfloat16)
    a2 = jnp.sum(a * a, axis=1, keepdims=True)
    cg_ref[...] = lax.broadcasted_iota(jnp.int32, (1, N_GT), 1)
    big = jnp.float32(3.0e38)
    bigi = jnp.int32(2 ** 30)

    @pl.when(i == 0)
    def _():
        g = gt_ref[...]
        b2 = g[0:1] * g[0:1] + g[1:2] * g[1:2] + g[2:3] * g[2:3]
        gaug_ref[0:3, :] = g
        gaug_ref[3:4, :] = -0.5 * b2

    aug1 = jnp.concatenate(
        [a_bf, jnp.ones((BM, 1), jnp.bfloat16)], axis=1)

    gmax = jnp.full((BM, 1), -big, jnp.float32)
    gidx = jnp.zeros((BM, 1), jnp.int32)
    for c in range(N_GT // CHUNK):
        ga = gaug_ref[:, c * CHUNK:(c + 1) * CHUNK]
        m = jnp.dot(aug1, ga.astype(jnp.bfloat16),
                    preferred_element_type=jnp.float32)
        cmax = jnp.max(m, axis=1, keepdims=True)
        csel = jnp.where(m == cmax, cg_ref[:, c * CHUNK:(c + 1) * CHUNK],
                         bigi)
        cidx = jnp.min(csel, axis=1, keepdims=True)
        upd = cmax > gmax
        gmax = jnp.where(upd, cmax, gmax)
        gidx = jnp.where(upd, cidx, gidx)
    idx_ref[...] = gidx
    minsq_ref[...] = jnp.maximum(a2 - 2.0 * gmax, 1e-12)

    rowg = lax.broadcasted_iota(jnp.int32, (BM, 1), 0) + i * BM

    ms = jnp.full((BM, 1), big, jnp.float32)
    for c in range(N_PRED // CHUNK):
        p = pt_ref[:, c * CHUNK:(c + 1) * CHUNK]
        b2 = p[0:1] * p[0:1] + p[1:2] * p[1:2] + p[2:3] * p[2:3]
        m = jnp.dot(a_bf, p.astype(jnp.bfloat16),
                    preferred_element_type=jnp.float32)
        r = b2 - 2.0 * m
        cg = cg_ref[:, c * CHUNK:(c + 1) * CHUNK]
        r = jnp.where(cg == rowg, big, r)
        ms = jnp.minimum(ms, jnp.min(r, axis=1, keepdims=True))
    minself_ref[...] = jnp.maximum(a2 + ms, 1e-12)


_k1 = pl.pallas_call(
    _k1_body,
    grid=(N_PRED // BM,),
    in_specs=[
        pl.BlockSpec((BM, 3), lambda i: (i, 0)),
        pl.BlockSpec((3, N_GT), lambda i: (0, 0)),
        pl.BlockSpec((3, N_PRED), lambda i: (0, 0)),
    ],
    out_specs=[
        pl.BlockSpec((BM, 1), lambda i: (i, 0)),
        pl.BlockSpec((BM, 1), lambda i: (i, 0)),
        pl.BlockSpec((BM, 1), lambda i: (i, 0)),
    ],
    out_shape=[
        jax.ShapeDtypeStruct((N_PRED, 1), jnp.float32),
        jax.ShapeDtypeStruct((N_PRED, 1), jnp.int32),
        jax.ShapeDtypeStruct((N_PRED, 1), jnp.float32),
    ],
    scratch_shapes=[pltpu.VMEM((1, N_GT), jnp.int32),
                    pltpu.VMEM((4, N_GT), jnp.float32)],
)


@functools.lru_cache(maxsize=1)
def _make_sc_gather():
    mesh = plsc.VectorSubcoreMesh(core_axis_name="c", subcore_axis_name="s")

    @functools.partial(
        pl.kernel,
        mesh=mesh,
        out_type=jax.ShapeDtypeStruct((N_PRED, 16), jnp.float32),
        scratch_types=[
            pltpu.VMEM((_B_PER_W,), jnp.int32),
            pltpu.VMEM((_B_PER_W, 16), jnp.float32),
            pltpu.SemaphoreType.DMA,
        ],
        compiler_params=pltpu.CompilerParams(use_tc_tiling_on_sc=False),
    )
    def gather_k(table_hbm, idx_hbm, out_hbm, idx_v, rows_v, sem):
        wid = lax.axis_index("s") * _SC_NC + lax.axis_index("c")
        base = wid * _B_PER_W
        pltpu.sync_copy(idx_hbm.at[pl.ds(base, _B_PER_W)], idx_v)
        pltpu.async_copy(table_hbm.at[idx_v], rows_v, sem).wait()
        pltpu.sync_copy(rows_v, out_hbm.at[pl.ds(base, _B_PER_W)])

    return gather_k


def _k3_body(minsq_ref, minself_ref, pnt_ref, gnt_ref, out_ref):
    minsq = minsq_ref[...]
    minself = minself_ref[...]
    pnt = pnt_ref[...]
    gnt = gnt_ref[...]

    attraction = jnp.sum(minsq) / (N_PRED * 3.0)

    d = jnp.sqrt(minself)
    x = 100.0 * (0.3 - d)
    pen = jnp.maximum(x, 0.0) + jnp.log(1.0 + jnp.exp(-jnp.abs(x)))
    repulsion = jnp.sum(pen * pen) / N_PRED

    pn2 = jnp.sum(pnt * pnt, axis=0, keepdims=True)
    gn2 = jnp.sum(gnt * gnt, axis=0, keepdims=True)
    pd = jnp.maximum(jnp.sqrt(pn2), 1e-5)
    gd = jnp.maximum(jnp.sqrt(gn2), 1e-5)
    dot = jnp.sum(pnt * gnt, axis=0, keepdims=True)
    cos = dot / (pd * gd)
    norm_loss = jnp.sum(1.0 - cos) / N_PRED

    out_ref[0, 0] = attraction + repulsion + 10.0 * norm_loss


_k3 = pl.pallas_call(
    _k3_body,
    out_specs=pl.BlockSpec(memory_space=pltpu.SMEM),
    out_shape=jax.ShapeDtypeStruct((1, 1), jnp.float32),
)


def kernel(pred_feat, pred_decoder, input_data, gt_data):
    pp = pred_feat[:, :3]
    pn = pred_feat[:, 3:]
    gp = gt_data[:, :3]
    gn = gt_data[:, 3:]

    minsq, idx, minself = _k1(pp, gp.T, pp.T)
    gtab = jnp.pad(gn, ((0, 0), (0, 13)))
    g = _make_sc_gather()(gtab, idx.reshape(N_PRED))
    out = _k3(minsq.reshape(32, 128), minself.reshape(32, 128),
              pn.T, g[:, :3].T)
    return out[0, 0]
